# Initial kernel scaffold; baseline (speedup 1.0000x reference)
import jax
import jax.numpy as jnp
from jax.experimental import pallas as pl


def kernel(x, edge_index, edge_attr, subgraph_nodes, W1, b1, g1, be1, W2, b2, g2, be2, W3, b3, g3, be3, Wc1, bc1, Wc2, bc2, Wc3, bc3):
    raise NotImplementedError("write your pallas kernel here")


# speedup vs baseline: 8.3434x; 8.3434x over previous
import functools

import jax
import jax.numpy as jnp
from jax import lax
from jax.experimental import pallas as pl
from jax.experimental.pallas import tpu as pltpu
from jax.experimental.pallas import tpu_sc as plsc

N = 10000
E = 320000
D = 128
H = 128
S = 512
K = 32
C = 10

NC = 2
NS = 16
L = 16
NW = NC * NS
EW = E // NW
KB = 80
NSUB = EW // KB
RPT = N // NS
NROW16 = N // L
SG_G = 4
SG_B = 128
SG_PW = S // NW

_mesh = plsc.VectorSubcoreMesh(core_axis_name="c", subcore_axis_name="s")


def _f32(shape):
    return jax.ShapeDtypeStruct(shape, jnp.float32)


def _bcast_lane(v16, t):
    idx = jnp.full((L,), t, jnp.int32)
    return lax.gather(
        v16, idx[:, None],
        lax.GatherDimensionNumbers(offset_dims=(), collapsed_slice_dims=(0,),
                                   start_index_map=(0,)),
        (1,), mode=lax.GatherScatterMode.PROMISE_IN_BOUNDS)


@functools.partial(
    pl.kernel,
    out_type=_f32((NC, N)),
    mesh=_mesh,
    scratch_types=[
        pltpu.VMEM((KB,), jnp.int32),
        pltpu.VMEM((KB,), jnp.float32),
        pltpu.VMEM((1000,), jnp.float32),
        pltpu.VMEM_SHARED((N,), jnp.float32),
    ],
)
def _deg_kernel(dst_hbm, ew_hbm, out_hbm, dst_row, ew_row, zeros_v, acc_sh):
    cid = lax.axis_index("c")
    sid = lax.axis_index("s")
    wid = sid * NC + cid

    def _zero(i, _):
        zeros_v[pl.ds(i * L, L)] = jnp.zeros((L,), jnp.float32)
        return 0

    lax.fori_loop(0, 1000 // L, _zero, 0)

    @pl.when(sid < N // 1000)
    def _():
        pltpu.sync_copy(zeros_v, acc_sh.at[pl.ds(sid * 1000, 1000)])

    plsc.subcore_barrier()

    def _scat(i, _):
        off = wid * EW + i * KB
        pltpu.sync_copy(dst_hbm.at[pl.ds(off, KB)], dst_row)
        pltpu.sync_copy(ew_hbm.at[pl.ds(off, KB)], ew_row)
        pltpu.sync_copy(ew_row, acc_sh.at[dst_row], add=True)
        return 0

    lax.fori_loop(0, NSUB, _scat, 0)
    plsc.subcore_barrier()

    @pl.when(sid == 0)
    def _():
        pltpu.sync_copy(acc_sh, out_hbm.at[cid])


@functools.partial(
    pl.kernel,
    out_type=_f32((NC, N, H)),
    mesh=_mesh,
    scratch_types=[
        pltpu.VMEM((KB,), jnp.int32),
        pltpu.VMEM((KB,), jnp.int32),
        pltpu.VMEM((KB,), jnp.float32),
        pltpu.VMEM((KB, H), jnp.float32),
        pltpu.VMEM_SHARED((N, H), jnp.float32),
        pltpu.SemaphoreType.DMA,
    ],
)
def _edge_kernel(hws_hbm, src_hbm, dst_hbm, ew_hbm, out_hbm,
                 src_row, dst_row, ew_row, rows_v, acc_sh, sem):
    cid = lax.axis_index("c")
    sid = lax.axis_index("s")
    wid = sid * NC + cid

    def _zero(r, _):
        for j in range(H // L):
            rows_v[r, pl.ds(j * L, L)] = jnp.zeros((L,), jnp.float32)
        return 0

    lax.fori_loop(0, KB, _zero, 0)

    @pl.when(sid < N // 1000)
    def _():
        base = sid * 1000
        for j in range(12):
            pltpu.sync_copy(rows_v.at[pl.ds(0, KB)],
                            acc_sh.at[pl.ds(base + j * KB, KB)])
        pltpu.sync_copy(rows_v.at[pl.ds(0, 40)],
                        acc_sh.at[pl.ds(base + 960, 40)])

    plsc.subcore_barrier()

    def _sub(i, _):
        off = wid * EW + i * KB
        pltpu.sync_copy(src_hbm.at[pl.ds(off, KB)], src_row)
        pltpu.sync_copy(dst_hbm.at[pl.ds(off, KB)], dst_row)
        pltpu.sync_copy(ew_hbm.at[pl.ds(off, KB)], ew_row)
        pltpu.async_copy(hws_hbm.at[src_row], rows_v, sem).wait()
        for g in range(KB // L):
            ew16 = ew_row[pl.ds(g * L, L)]
            for t in range(L):
                bc = _bcast_lane(ew16, t)
                e = g * L + t
                for j in range(H // L):
                    rows_v[e, pl.ds(j * L, L)] = rows_v[e, pl.ds(j * L, L)] * bc
        pltpu.sync_copy(rows_v, acc_sh.at[dst_row], add=True)
        return 0

    lax.fori_loop(0, NSUB, _sub, 0)
    plsc.subcore_barrier()

    @pl.when(sid < N // 1000)
    def _():
        wbase = sid * 1000
        pltpu.sync_copy(acc_sh.at[pl.ds(wbase, 1000)],
                        out_hbm.at[cid, pl.ds(wbase, 1000)])


@functools.partial(
    pl.kernel,
    out_type=_f32((S, 3 * H)),
    mesh=_mesh,
    scratch_types=[
        pltpu.VMEM((SG_G, SG_B), jnp.int32),
        pltpu.VMEM((SG_B, H), jnp.float32),
        pltpu.VMEM((SG_PW, 3 * H), jnp.float32),
        pltpu.SemaphoreType.DMA,
    ],
)
def _readout_kernel(h_hbm, sg_hbm, out_hbm, idx_v, rows_v, emb_v, sem):
    cid = lax.axis_index("c")
    sid = lax.axis_index("s")
    wid = sid * NC + cid

    pltpu.sync_copy(sg_hbm.at[wid], idx_v)
    for g in range(SG_G):
        pltpu.async_copy(h_hbm.at[idx_v.at[g]], rows_v, sem).wait()
        for t in range(SG_B // K):
            si = g * (SG_B // K) + t
            for j in range(H // L):
                init_s = rows_v[t * K, pl.ds(j * L, L)]

                def _red(r, sm):
                    v = rows_v[t * K + r, pl.ds(j * L, L)]
                    return (sm[0] + v, jnp.maximum(sm[1], v))

                ssum, smax = lax.fori_loop(1, K, _red, (init_s, init_s))
                emb_v[si, pl.ds(j * L, L)] = ssum * (1.0 / K)
                emb_v[si, pl.ds(H + j * L, L)] = smax
                emb_v[si, pl.ds(2 * H + j * L, L)] = ssum
    pltpu.sync_copy(emb_v, out_hbm.at[pl.ds(wid * SG_PW, SG_PW)])


def _dis_body(degp_ref, o_ref):
    d = jnp.sum(degp_ref[...], axis=0, keepdims=True) + 1.0
    o_ref[...] = lax.rsqrt(d)


def _mm_scale_body(x_ref, w_ref, dis_ref, o_ref):
    o_ref[...] = jnp.dot(x_ref[...] * dis_ref[...], w_ref[...],
                         preferred_element_type=jnp.float32)


def _bn_relu(p_ref, hws_ref, dis_ref, b_ref, g_ref, be_ref):
    acc = p_ref[0] + p_ref[1] + hws_ref[...]
    out = acc * dis_ref[...] + b_ref[...]
    mu = jnp.mean(out, axis=0, keepdims=True)
    xc = out - mu
    var = jnp.mean(xc * xc, axis=0, keepdims=True)
    return jnp.maximum(xc * lax.rsqrt(var + 1e-5) * g_ref[...] + be_ref[...],
                       0.0)


def _mid_body(p_ref, hws_ref, dis_ref, b_ref, g_ref, be_ref, w_ref, o_ref):
    h = _bn_relu(p_ref, hws_ref, dis_ref, b_ref, g_ref, be_ref)
    o_ref[...] = jnp.dot(h * dis_ref[...], w_ref[...],
                         preferred_element_type=jnp.float32)


def _last_body(p_ref, hws_ref, dis_ref, b_ref, g_ref, be_ref, o_ref):
    o_ref[...] = _bn_relu(p_ref, hws_ref, dis_ref, b_ref, g_ref, be_ref)


def _cls_body(emb_ref, w1_ref, b1_ref, w2_ref, b2_ref, w3_ref, b3_ref, o_ref):
    z = jnp.maximum(jnp.dot(emb_ref[...], w1_ref[...],
                            preferred_element_type=jnp.float32) + b1_ref[...], 0.0)
    z = jnp.maximum(jnp.dot(z, w2_ref[...],
                            preferred_element_type=jnp.float32) + b2_ref[...], 0.0)
    o_ref[...] = jnp.dot(z, w3_ref[...],
                         preferred_element_type=jnp.float32) + b3_ref[...]


_dis_call = pl.pallas_call(_dis_body, out_shape=_f32((1, N)))
_mm_scale_call = pl.pallas_call(_mm_scale_body, out_shape=_f32((N, H)))
_mid_call = pl.pallas_call(_mid_body, out_shape=_f32((N, H)))
_last_call = pl.pallas_call(_last_body, out_shape=_f32((N, H)))
_cls_call = pl.pallas_call(_cls_body, out_shape=_f32((S, 128)))


def kernel(x, edge_index, edge_attr, subgraph_nodes,
           W1, b1, g1, be1, W2, b2, g2, be2, W3, b3, g3, be3,
           Wc1, bc1, Wc2, bc2, Wc3, bc3):
    ei = edge_index.astype(jnp.int32)
    src1 = ei[0]
    dst1 = ei[1]
    sg3 = subgraph_nodes.astype(jnp.int32).reshape(NW, SG_G, SG_B)

    degp = _deg_kernel(dst1, edge_attr)
    dis = _dis_call(degp).reshape(N, 1)

    hws = _mm_scale_call(x, W1, dis)
    for (b, g, be, Wn) in ((b1, g1, be1, W2), (b2, g2, be2, W3)):
        p = _edge_kernel(hws, src1, dst1, edge_attr)
        hws = _mid_call(p, hws, dis, b.reshape(1, H), g.reshape(1, H),
                        be.reshape(1, H), Wn)
    p = _edge_kernel(hws, src1, dst1, edge_attr)
    h3 = _last_call(p, hws, dis, b3.reshape(1, H), g3.reshape(1, H),
                    be3.reshape(1, H))

    emb = _readout_kernel(h3, sg3)

    pad2 = jnp.zeros((H, 128 - H // 2), jnp.float32)
    w2p = jnp.concatenate([Wc2, pad2], axis=1)
    b2p = jnp.concatenate([bc2, jnp.zeros((128 - H // 2,), jnp.float32)]).reshape(1, 128)
    pad3r = jnp.zeros((128 - H // 2, C), jnp.float32)
    w3r = jnp.concatenate([Wc3, pad3r], axis=0)
    pad3c = jnp.zeros((128, 128 - C), jnp.float32)
    w3p = jnp.concatenate([w3r, pad3c], axis=1)
    b3p = jnp.concatenate([bc3, jnp.zeros((128 - C,), jnp.float32)]).reshape(1, 128)

    logits = _cls_call(emb, Wc1, bc1.reshape(1, H), w2p, b2p, w3p, b3p)
    return logits[:, :C]
